# baseline (speedup 1.0000x reference)
import jax
import jax.numpy as jnp
from jax import lax
from jax.experimental import pallas as pl
from jax.experimental.pallas import tpu as pltpu, tpu_sc as plsc

RHO = 70.0
T, N = 1025, 20000
NPOS = 65536
SPAIRS = 100
TM1 = T - 1
NW = 32
PP = NPOS // NW
RT = TM1 // NW
SS = RT * SPAIRS
L = 16
SSP = SS + L
NVR = (SPAIRS + L - 1) // L


def _sigmoid(z):
    return 1.0 / (1.0 + jnp.exp(-z))


def _body(x_hbm, th_hbm, tp_hbm, up_hbm, vp_hbm, us_hbm, vs_hbm,
          kpos_hbm, kneg_hbm,
          th_v, tp_v, up_v, vp_v, iu_v, iv_v, gu_v, gv_v,
          su_v, sv_v, siu_v, siv_v, sgu_v, sgv_v, op_v, on_v, sem):
    wid = lax.axis_index("s") * 2 + lax.axis_index("c")

    pltpu.sync_copy(th_hbm, th_v)
    eps = _sigmoid(th_v[...]) * 0.5

    base = wid * PP
    pltpu.sync_copy(tp_hbm.at[pl.ds(base, PP)], tp_v)
    pltpu.sync_copy(up_hbm.at[pl.ds(base, PP)], up_v)
    pltpu.sync_copy(vp_hbm.at[pl.ds(base, PP)], vp_v)
    pltpu.sync_copy(us_hbm.at[pl.ds(wid * SS, SS)], su_v.at[pl.ds(0, SS)])
    pltpu.sync_copy(vs_hbm.at[pl.ds(wid * SS, SS)], sv_v.at[pl.ds(0, SS)])

    def pos_idx(k, c):
        sl = pl.ds(k * L, L)
        roff = tp_v[sl] * N
        iu_v[sl] = roff + up_v[sl]
        iv_v[sl] = roff + vp_v[sl]
        return c
    lax.fori_loop(0, PP // L, pos_idx, 0)

    iota = lax.iota(jnp.int32, L)
    row0 = wid * RT

    def samp_idx(rr, c):
        roff = (row0 + rr) * N

        def one(q, c2):
            sl = pl.ds(rr * SPAIRS + q * L, L)
            cu_ = jnp.minimum(jnp.maximum(su_v[sl], 0), N - 1)
            cv_ = jnp.minimum(jnp.maximum(sv_v[sl], 0), N - 1)
            siu_v[sl] = cu_ + roff
            siv_v[sl] = cv_ + roff
            return c2
        return lax.fori_loop(0, NVR, one, c)
    lax.fori_loop(0, RT, samp_idx, 0)

    cu = pltpu.async_copy(x_hbm.at[iu_v], gu_v, sem)
    cv = pltpu.async_copy(x_hbm.at[iv_v], gv_v, sem)
    gsu = pltpu.async_copy(x_hbm.at[siu_v.at[pl.ds(0, SS)]],
                           sgu_v.at[pl.ds(0, SS)], sem)
    gsv = pltpu.async_copy(x_hbm.at[siv_v.at[pl.ds(0, SS)]],
                           sgv_v.at[pl.ds(0, SS)], sem)

    cu.wait()
    cv.wait()

    def pos_kap(k, c):
        sl = pl.ds(k * L, L)
        d = gu_v[sl] - gv_v[sl]
        op_v[sl] = _sigmoid(RHO * (eps - jnp.abs(d)))
        return c
    lax.fori_loop(0, PP // L, pos_kap, 0)
    pltpu.sync_copy(op_v, kpos_hbm.at[pl.ds(base, PP)])

    gsu.wait()
    gsv.wait()

    def samp_kap(rr, carry):
        on_a, on_b = carry

        def one(q, acc):
            sl = pl.ds(rr * SPAIRS + q * L, L)
            d = sgu_v[sl] - sgv_v[sl]
            kap = _sigmoid(RHO * (eps - jnp.abs(d)))
            return acc + jnp.where(iota < SPAIRS - q * L, kap, 0.0)
        acc = lax.fori_loop(0, NVR, one, jnp.zeros((L,), jnp.float32))
        kn = 1.0 - jnp.sum(acc) * (1.0 / SPAIRS)
        lane = jnp.where(iota == (rr & (L - 1)), kn, 0.0)
        in_a = jnp.where(rr < L, lane, 0.0)
        return (on_a + in_a, on_b + (lane - in_a))

    zero = jnp.zeros((L,), jnp.float32)
    on_a, on_b = lax.fori_loop(0, RT, samp_kap, (zero, zero))

    on_v[pl.ds(0, L)] = on_a
    on_v[pl.ds(L, L)] = on_b
    pltpu.sync_copy(on_v, kneg_hbm.at[pl.ds(wid * RT, RT)])


def kernel(X, theta, u_pos, v_pos, t_pos, u_sample, v_sample):
    x_flat = X.reshape(-1)
    th16 = jnp.broadcast_to(theta.astype(jnp.float32), (L,))
    us_f = u_sample.reshape(-1)
    vs_f = v_sample.reshape(-1)

    mesh = plsc.VectorSubcoreMesh(core_axis_name="c", subcore_axis_name="s")
    run = pl.kernel(
        _body,
        out_type=(
            jax.ShapeDtypeStruct((NPOS,), jnp.float32),
            jax.ShapeDtypeStruct((TM1,), jnp.float32),
        ),
        mesh=mesh,
        compiler_params=pltpu.CompilerParams(
            use_tc_tiling_on_sc=False, needs_layout_passes=False),
        scratch_types=[
            pltpu.VMEM((L,), jnp.float32),
            pltpu.VMEM((PP,), jnp.int32),
            pltpu.VMEM((PP,), jnp.int32),
            pltpu.VMEM((PP,), jnp.int32),
            pltpu.VMEM((PP,), jnp.int32),
            pltpu.VMEM((PP,), jnp.int32),
            pltpu.VMEM((PP,), jnp.float32),
            pltpu.VMEM((PP,), jnp.float32),
            pltpu.VMEM((SSP,), jnp.int32),
            pltpu.VMEM((SSP,), jnp.int32),
            pltpu.VMEM((SSP,), jnp.int32),
            pltpu.VMEM((SSP,), jnp.int32),
            pltpu.VMEM((SSP,), jnp.float32),
            pltpu.VMEM((SSP,), jnp.float32),
            pltpu.VMEM((PP,), jnp.float32),
            pltpu.VMEM((RT,), jnp.float32),
            pltpu.SemaphoreType.DMA,
        ],
    )
    kappa_pos, kappa_neg = run(x_flat, th16, t_pos, u_pos, v_pos, us_f, vs_f)
    return kappa_pos, kappa_neg


# speedup vs baseline: 1.0194x; 1.0194x over previous
import jax
import jax.numpy as jnp
from jax import lax
from jax.experimental import pallas as pl
from jax.experimental.pallas import tpu as pltpu, tpu_sc as plsc

RHO = 70.0
T, N = 1025, 20000
NPOS = 65536
SPAIRS = 100
TM1 = T - 1
NW = 32
PP = NPOS // NW
RT = TM1 // NW
SS = RT * SPAIRS
L = 16


def _sigmoid(z):
    return 1.0 / (1.0 + jnp.exp(-z))


def _body(x_hbm, th_hbm, tp_hbm, up_hbm, vp_hbm, us_hbm, vs_hbm,
          kpos_hbm, kneg_hbm,
          th_v, tp_v, up_v, vp_v, iu_v, iv_v, gu_v, gv_v,
          su_v, sv_v, siu_v, siv_v, sgu_v, sgv_v, op_v, on_v, sem):
    wid = lax.axis_index("s") * 2 + lax.axis_index("c")

    pltpu.sync_copy(th_hbm, th_v)
    eps = _sigmoid(th_v[...]) * 0.5

    base = wid * PP
    pltpu.sync_copy(tp_hbm.at[pl.ds(base, PP)], tp_v)
    pltpu.sync_copy(up_hbm.at[pl.ds(base, PP)], up_v)
    pltpu.sync_copy(vp_hbm.at[pl.ds(base, PP)], vp_v)
    pltpu.sync_copy(us_hbm.at[wid], su_v)
    pltpu.sync_copy(vs_hbm.at[wid], sv_v)

    def pos_idx(k, c):
        sl = pl.ds(k * L, L)
        roff = tp_v[sl] * N
        iu_v[sl] = roff + up_v[sl]
        iv_v[sl] = roff + vp_v[sl]
        return c
    lax.fori_loop(0, PP // L, pos_idx, 0)

    iota = lax.iota(jnp.int32, L)
    t0 = (wid * RT + iota) * N
    t1 = (wid * RT + L + iota) * N

    def samp_idx(j, c):
        b = j * 2 * L
        s0 = pl.ds(b, L)
        s1 = pl.ds(b + L, L)
        siu_v[s0] = su_v[s0] + t0
        siu_v[s1] = su_v[s1] + t1
        siv_v[s0] = sv_v[s0] + t0
        siv_v[s1] = sv_v[s1] + t1
        return c
    lax.fori_loop(0, SPAIRS, samp_idx, 0)

    cu = pltpu.async_copy(x_hbm.at[iu_v], gu_v, sem)
    cv = pltpu.async_copy(x_hbm.at[iv_v], gv_v, sem)
    gsu = pltpu.async_copy(x_hbm.at[siu_v], sgu_v, sem)
    gsv = pltpu.async_copy(x_hbm.at[siv_v], sgv_v, sem)

    cu.wait()
    cv.wait()

    def pos_kap(k, c):
        sl = pl.ds(k * L, L)
        d = gu_v[sl] - gv_v[sl]
        op_v[sl] = _sigmoid(RHO * (eps - jnp.abs(d)))
        return c
    lax.fori_loop(0, PP // L, pos_kap, 0)
    pltpu.sync_copy(op_v, kpos_hbm.at[pl.ds(base, PP)])

    gsu.wait()
    gsv.wait()

    def samp_kap(j, acc):
        a0, a1 = acc
        b = j * 2 * L
        s0 = pl.ds(b, L)
        s1 = pl.ds(b + L, L)
        d0 = sgu_v[s0] - sgv_v[s0]
        d1 = sgu_v[s1] - sgv_v[s1]
        a0 = a0 + _sigmoid(RHO * (eps - jnp.abs(d0)))
        a1 = a1 + _sigmoid(RHO * (eps - jnp.abs(d1)))
        return (a0, a1)
    zero = jnp.zeros((L,), jnp.float32)
    a0, a1 = lax.fori_loop(0, SPAIRS, samp_kap, (zero, zero))

    on_v[pl.ds(0, L)] = 1.0 - a0 * (1.0 / SPAIRS)
    on_v[pl.ds(L, L)] = 1.0 - a1 * (1.0 / SPAIRS)
    pltpu.sync_copy(on_v, kneg_hbm.at[pl.ds(wid * RT, RT)])


def kernel(X, theta, u_pos, v_pos, t_pos, u_sample, v_sample):
    x_flat = X.reshape(-1)
    th16 = jnp.broadcast_to(theta.astype(jnp.float32), (L,))
    us_p = u_sample.reshape(NW, RT, SPAIRS).transpose(0, 2, 1).reshape(NW, SS)
    vs_p = v_sample.reshape(NW, RT, SPAIRS).transpose(0, 2, 1).reshape(NW, SS)

    mesh = plsc.VectorSubcoreMesh(core_axis_name="c", subcore_axis_name="s")
    run = pl.kernel(
        _body,
        out_type=(
            jax.ShapeDtypeStruct((NPOS,), jnp.float32),
            jax.ShapeDtypeStruct((TM1,), jnp.float32),
        ),
        mesh=mesh,
        compiler_params=pltpu.CompilerParams(
            use_tc_tiling_on_sc=False, needs_layout_passes=False),
        scratch_types=[
            pltpu.VMEM((L,), jnp.float32),
            pltpu.VMEM((PP,), jnp.int32),
            pltpu.VMEM((PP,), jnp.int32),
            pltpu.VMEM((PP,), jnp.int32),
            pltpu.VMEM((PP,), jnp.int32),
            pltpu.VMEM((PP,), jnp.int32),
            pltpu.VMEM((PP,), jnp.float32),
            pltpu.VMEM((PP,), jnp.float32),
            pltpu.VMEM((SS,), jnp.int32),
            pltpu.VMEM((SS,), jnp.int32),
            pltpu.VMEM((SS,), jnp.int32),
            pltpu.VMEM((SS,), jnp.int32),
            pltpu.VMEM((SS,), jnp.float32),
            pltpu.VMEM((SS,), jnp.float32),
            pltpu.VMEM((PP,), jnp.float32),
            pltpu.VMEM((RT,), jnp.float32),
            pltpu.SemaphoreType.DMA,
        ],
    )
    kappa_pos, kappa_neg = run(x_flat, th16, t_pos, u_pos, v_pos, us_p, vs_p)
    return kappa_pos, kappa_neg


# speedup vs baseline: 1.0236x; 1.0041x over previous
import jax
import jax.numpy as jnp
from jax import lax
from jax.experimental import pallas as pl
from jax.experimental.pallas import tpu as pltpu, tpu_sc as plsc

RHO = 70.0
T, N = 1025, 20000
NPOS = 65536
SPAIRS = 100
TM1 = T - 1
NW = 32
PP = NPOS // NW
RT = TM1 // NW
SS = RT * SPAIRS
L = 16


def _sigmoid(z):
    return 1.0 / (1.0 + jnp.exp(-z))


def _body(x_hbm, th_hbm, tp_hbm, up_hbm, vp_hbm, us_hbm, vs_hbm,
          kpos_hbm, kneg_hbm,
          th_v, tp_v, up_v, vp_v, iu_v, iv_v, gu_v, gv_v,
          su_v, sv_v, siu_v, siv_v, sgu_v, sgv_v, op_v, on_v, sem):
    wid = lax.axis_index("s") * 2 + lax.axis_index("c")

    pltpu.sync_copy(th_hbm, th_v)
    eps = _sigmoid(th_v[...]) * 0.5

    base = wid * PP
    pltpu.sync_copy(tp_hbm.at[pl.ds(base, PP)], tp_v)
    pltpu.sync_copy(up_hbm.at[pl.ds(base, PP)], up_v)
    pltpu.sync_copy(vp_hbm.at[pl.ds(base, PP)], vp_v)
    pltpu.sync_copy(us_hbm.at[wid], su_v)
    pltpu.sync_copy(vs_hbm.at[wid], sv_v)

    def pos_idx(k, c):
        s0 = pl.ds(2 * k * L, L)
        s1 = pl.ds((2 * k + 1) * L, L)
        r0 = tp_v[s0] * N
        r1 = tp_v[s1] * N
        iu_v[s0] = r0 + up_v[s0]
        iv_v[s0] = r0 + vp_v[s0]
        iu_v[s1] = r1 + up_v[s1]
        iv_v[s1] = r1 + vp_v[s1]
        return c
    lax.fori_loop(0, PP // L // 2, pos_idx, 0)

    iota = lax.iota(jnp.int32, L)
    t0 = (wid * RT + iota) * N
    t1 = (wid * RT + L + iota) * N

    def samp_idx(j2, c):
        b = j2 * 4 * L
        s0 = pl.ds(b, L)
        s1 = pl.ds(b + L, L)
        s2 = pl.ds(b + 2 * L, L)
        s3 = pl.ds(b + 3 * L, L)
        siu_v[s0] = su_v[s0] + t0
        siu_v[s1] = su_v[s1] + t1
        siv_v[s0] = sv_v[s0] + t0
        siv_v[s1] = sv_v[s1] + t1
        siu_v[s2] = su_v[s2] + t0
        siu_v[s3] = su_v[s3] + t1
        siv_v[s2] = sv_v[s2] + t0
        siv_v[s3] = sv_v[s3] + t1
        return c
    lax.fori_loop(0, SPAIRS // 2, samp_idx, 0)

    cu = pltpu.async_copy(x_hbm.at[iu_v], gu_v, sem)
    cv = pltpu.async_copy(x_hbm.at[iv_v], gv_v, sem)
    gsu = pltpu.async_copy(x_hbm.at[siu_v], sgu_v, sem)
    gsv = pltpu.async_copy(x_hbm.at[siv_v], sgv_v, sem)

    cu.wait()
    cv.wait()

    def pos_kap(k, c):
        s0 = pl.ds(2 * k * L, L)
        s1 = pl.ds((2 * k + 1) * L, L)
        d0 = gu_v[s0] - gv_v[s0]
        d1 = gu_v[s1] - gv_v[s1]
        op_v[s0] = _sigmoid(RHO * (eps - jnp.abs(d0)))
        op_v[s1] = _sigmoid(RHO * (eps - jnp.abs(d1)))
        return c
    lax.fori_loop(0, PP // L // 2, pos_kap, 0)
    pltpu.sync_copy(op_v, kpos_hbm.at[pl.ds(base, PP)])

    gsu.wait()
    gsv.wait()

    def samp_kap(j2, acc):
        a0, a1 = acc
        b = j2 * 4 * L
        s0 = pl.ds(b, L)
        s1 = pl.ds(b + L, L)
        s2 = pl.ds(b + 2 * L, L)
        s3 = pl.ds(b + 3 * L, L)
        d0 = sgu_v[s0] - sgv_v[s0]
        d1 = sgu_v[s1] - sgv_v[s1]
        d2 = sgu_v[s2] - sgv_v[s2]
        d3 = sgu_v[s3] - sgv_v[s3]
        a0 = a0 + _sigmoid(RHO * (eps - jnp.abs(d0)))
        a1 = a1 + _sigmoid(RHO * (eps - jnp.abs(d1)))
        a0 = a0 + _sigmoid(RHO * (eps - jnp.abs(d2)))
        a1 = a1 + _sigmoid(RHO * (eps - jnp.abs(d3)))
        return (a0, a1)
    zero = jnp.zeros((L,), jnp.float32)
    a0, a1 = lax.fori_loop(0, SPAIRS // 2, samp_kap, (zero, zero))

    on_v[pl.ds(0, L)] = 1.0 - a0 * (1.0 / SPAIRS)
    on_v[pl.ds(L, L)] = 1.0 - a1 * (1.0 / SPAIRS)
    pltpu.sync_copy(on_v, kneg_hbm.at[pl.ds(wid * RT, RT)])


def kernel(X, theta, u_pos, v_pos, t_pos, u_sample, v_sample):
    x_flat = X.reshape(-1)
    th16 = jnp.broadcast_to(theta.astype(jnp.float32), (L,))
    us_p = u_sample.reshape(NW, RT, SPAIRS).transpose(0, 2, 1).reshape(NW, SS)
    vs_p = v_sample.reshape(NW, RT, SPAIRS).transpose(0, 2, 1).reshape(NW, SS)

    mesh = plsc.VectorSubcoreMesh(core_axis_name="c", subcore_axis_name="s")
    run = pl.kernel(
        _body,
        out_type=(
            jax.ShapeDtypeStruct((NPOS,), jnp.float32),
            jax.ShapeDtypeStruct((TM1,), jnp.float32),
        ),
        mesh=mesh,
        compiler_params=pltpu.CompilerParams(
            use_tc_tiling_on_sc=False, needs_layout_passes=False),
        scratch_types=[
            pltpu.VMEM((L,), jnp.float32),
            pltpu.VMEM((PP,), jnp.int32),
            pltpu.VMEM((PP,), jnp.int32),
            pltpu.VMEM((PP,), jnp.int32),
            pltpu.VMEM((PP,), jnp.int32),
            pltpu.VMEM((PP,), jnp.int32),
            pltpu.VMEM((PP,), jnp.float32),
            pltpu.VMEM((PP,), jnp.float32),
            pltpu.VMEM((SS,), jnp.int32),
            pltpu.VMEM((SS,), jnp.int32),
            pltpu.VMEM((SS,), jnp.int32),
            pltpu.VMEM((SS,), jnp.int32),
            pltpu.VMEM((SS,), jnp.float32),
            pltpu.VMEM((SS,), jnp.float32),
            pltpu.VMEM((PP,), jnp.float32),
            pltpu.VMEM((RT,), jnp.float32),
            pltpu.SemaphoreType.DMA,
        ],
    )
    kappa_pos, kappa_neg = run(x_flat, th16, t_pos, u_pos, v_pos, us_p, vs_p)
    return kappa_pos, kappa_neg
